# baseline (speedup 1.0000x reference)
import functools

import jax
import jax.numpy as jnp
from jax import lax
from jax.experimental import pallas as pl
from jax.experimental.pallas import tpu as pltpu
from jax.experimental.pallas import tpu_sc as plsc

N, E, D, H = 10000, 160000, 1024, 128

NC, NS = 2, 16
NW = NC * NS
CHUNK = 80
EPAD = 163840
CPW = EPAD // NW
NCHUNK = CPW // CHUNK
NPAD = 10240
HA = H + 16
OCH = CHUNK
RB = 400
GRID = N // RB

_MESH = plsc.VectorSubcoreMesh(core_axis_name="c", subcore_axis_name="s",
                               num_cores=NC, num_subcores=NS)
NBUF = 2


def _sc_segsum(ha, src2, dst2, zrow):
    @functools.partial(
        pl.kernel,
        out_type=jax.ShapeDtypeStruct((NC, NPAD, HA), jnp.float32),
        mesh=_MESH,
        compiler_params=pltpu.CompilerParams(use_tc_tiling_on_sc=False),
        scratch_types=[
            pltpu.VMEM_SHARED((NPAD, HA), jnp.float32),
            pltpu.VMEM((NCHUNK, CHUNK), jnp.int32),
            pltpu.VMEM((NCHUNK, CHUNK), jnp.int32),
        ] + [pltpu.VMEM((CHUNK, HA), jnp.float32) for _ in range(NBUF)]
          + [pltpu.SemaphoreType.DMA for _ in range(NBUF)],
    )
    def k(h_hbm, src_hbm, dst_hbm, zrow_hbm, agg_out, agg_sh, src_v, dst_v,
          *bufsem):
        bufs, sems = bufsem[:NBUF], bufsem[NBUF:]
        cid = lax.axis_index("c")
        sid = lax.axis_index("s")
        wid = sid * NC + cid
        rpt = NPAD // NS
        r0 = sid * rpt
        cbase = wid * NCHUNK
        pltpu.sync_copy(src_hbm.at[pl.ds(cbase, NCHUNK)], src_v)
        pltpu.sync_copy(dst_hbm.at[pl.ds(cbase, NCHUNK)], dst_v)
        pltpu.sync_copy(zrow_hbm, bufs[0])
        for j in range(rpt // OCH):
            pltpu.sync_copy(bufs[0], agg_sh.at[pl.ds(r0 + j * OCH, OCH)])
        for b in range(NBUF):
            pltpu.async_copy(h_hbm.at[src_v.at[b]], bufs[b], sems[b])
        plsc.subcore_barrier()

        def round_(g, carry):
            for b in range(NBUF):
                i = g * NBUF + b
                pltpu.make_async_copy(h_hbm.at[src_v.at[i]], bufs[b],
                                      sems[b]).wait()
                pltpu.sync_copy(bufs[b], agg_sh.at[dst_v.at[i]], add=True)
                j = i + NBUF

                @pl.when(j < NCHUNK)
                def _():
                    pltpu.async_copy(h_hbm.at[src_v.at[j]], bufs[b], sems[b])
            return carry

        lax.fori_loop(0, NCHUNK // NBUF, round_, 0)
        plsc.subcore_barrier()
        nout = rpt // OCH
        for j in range(nout):
            b = j % NBUF
            sl = pl.ds(r0 + j * OCH, OCH)
            if j >= NBUF:
                psl = pl.ds(r0 + (j - NBUF) * OCH, OCH)
                pltpu.make_async_copy(bufs[b], agg_out.at[cid, psl],
                                      sems[b]).wait()
            pltpu.sync_copy(agg_sh.at[sl], bufs[b])
            pltpu.async_copy(bufs[b], agg_out.at[cid, sl], sems[b])
        for j in range(nout - NBUF, nout):
            b = j % NBUF
            sl = pl.ds(r0 + j * OCH, OCH)
            pltpu.make_async_copy(bufs[b], agg_out.at[cid, sl],
                                  sems[b]).wait()

    return k(ha, src2, dst2, zrow)


def _ln_relu(x, g, b):
    m = jnp.mean(x, axis=-1, keepdims=True)
    v = jnp.mean((x - m) ** 2, axis=-1, keepdims=True)
    return jnp.maximum((x - m) * lax.rsqrt(v + 1e-5) * g + b, 0.0)


def _aug(h):
    return jnp.concatenate(
        [h, jnp.ones((h.shape[0], HA - H), jnp.float32)], axis=-1)


def _proj_body(x_ref, w_ref, b_ref, g_ref, be_ref, o_ref):
    h = jnp.dot(x_ref[...], w_ref[...], preferred_element_type=jnp.float32)
    o_ref[...] = _aug(_ln_relu(h + b_ref[...], g_ref[...], be_ref[...]))


def _proj(x, w, b, g, be):
    return pl.pallas_call(
        _proj_body,
        grid=(GRID,),
        in_specs=[
            pl.BlockSpec((RB, D), lambda i: (i, 0)),
            pl.BlockSpec((D, H), lambda i: (0, 0)),
            pl.BlockSpec((1, H), lambda i: (0, 0)),
            pl.BlockSpec((1, H), lambda i: (0, 0)),
            pl.BlockSpec((1, H), lambda i: (0, 0)),
        ],
        out_specs=pl.BlockSpec((RB, HA), lambda i: (i, 0)),
        out_shape=jax.ShapeDtypeStruct((N, HA), jnp.float32),
    )(x, w, b, g, be)


def _sage_block(p_ref, h_ref, wl_ref, bl_ref, wr_ref, g_ref, be_ref):
    pa = p_ref[0] + p_ref[1]
    agg = pa[:, :H]
    cnt = pa[:, H:H + 1]
    mean = agg / jnp.maximum(cnt, 1.0)
    h = h_ref[...][:, :H]
    hn = (jnp.dot(mean, wl_ref[...], preferred_element_type=jnp.float32)
          + bl_ref[...]
          + jnp.dot(h, wr_ref[...], preferred_element_type=jnp.float32))
    return h + _ln_relu(hn, g_ref[...], be_ref[...])


def _combine_body(p_ref, h_ref, wl_ref, bl_ref, wr_ref, g_ref, be_ref, o_ref):
    o_ref[...] = _aug(
        _sage_block(p_ref, h_ref, wl_ref, bl_ref, wr_ref, g_ref, be_ref))


_SAGE_SPECS = [
    pl.BlockSpec((NC, RB, HA), lambda i: (0, i, 0)),
    pl.BlockSpec((RB, HA), lambda i: (i, 0)),
    pl.BlockSpec((H, H), lambda i: (0, 0)),
    pl.BlockSpec((1, H), lambda i: (0, 0)),
    pl.BlockSpec((H, H), lambda i: (0, 0)),
    pl.BlockSpec((1, H), lambda i: (0, 0)),
    pl.BlockSpec((1, H), lambda i: (0, 0)),
]


def _combine(p, h, wl, bl, wr, g, be):
    return pl.pallas_call(
        _combine_body,
        grid=(GRID,),
        in_specs=_SAGE_SPECS,
        out_specs=pl.BlockSpec((RB, HA), lambda i: (i, 0)),
        out_shape=jax.ShapeDtypeStruct((N, HA), jnp.float32),
    )(p, h, wl, bl, wr, g, be)


def _final_body(p_ref, h_ref, wl_ref, bl_ref, wr_ref, g_ref, be_ref,
                w1_ref, b1_ref, g3_ref, be3_ref, w2_ref, b2_ref,
                o_ref, sum_sc, max_sc):
    i = pl.program_id(0)
    h2 = _sage_block(p_ref, h_ref, wl_ref, bl_ref, wr_ref, g_ref,
                     be_ref)
    blk = h2.reshape(RB // 8, 8, H)
    bsum = jnp.sum(blk, axis=0)
    bmax = jnp.max(blk, axis=0)

    @pl.when(i == 0)
    def _():
        sum_sc[...] = bsum
        max_sc[...] = bmax

    @pl.when(i > 0)
    def _():
        sum_sc[...] = sum_sc[...] + bsum
        max_sc[...] = jnp.maximum(max_sc[...], bmax)

    @pl.when(i == pl.num_programs(0) - 1)
    def _():
        hm = jnp.sum(sum_sc[...], axis=0, keepdims=True) / N
        hx = jnp.max(max_sc[...], axis=0, keepdims=True)
        r = jnp.concatenate([hm, hx], axis=-1)
        r8 = jnp.broadcast_to(r, (8, 2 * H))
        r8 = jnp.dot(r8, w1_ref[...], preferred_element_type=jnp.float32)
        r8 = _ln_relu(r8 + b1_ref[...], g3_ref[...], be3_ref[...])
        out8 = (jnp.dot(r8, w2_ref[...], preferred_element_type=jnp.float32)
                + b2_ref[...])
        o_ref[...] = out8[:1]


def _final(p, h, wl, bl, wr, g, be, w1, b1, g3, be3, w2, b2):
    return pl.pallas_call(
        _final_body,
        grid=(GRID,),
        in_specs=_SAGE_SPECS + [
            pl.BlockSpec((2 * H, H), lambda i: (0, 0)),
            pl.BlockSpec((1, H), lambda i: (0, 0)),
            pl.BlockSpec((1, H), lambda i: (0, 0)),
            pl.BlockSpec((1, H), lambda i: (0, 0)),
            pl.BlockSpec((H, H), lambda i: (0, 0)),
            pl.BlockSpec((1, H), lambda i: (0, 0)),
        ],
        out_specs=pl.BlockSpec((1, H), lambda i: (0, 0)),
        out_shape=jax.ShapeDtypeStruct((1, H), jnp.float32),
        scratch_shapes=[pltpu.VMEM((8, H), jnp.float32),
                        pltpu.VMEM((8, H), jnp.float32)],
    )(p, h, wl, bl, wr, g, be, w1, b1, g3, be3, w2, b2)


def kernel(x, edge_index, W0, b0, g0, be0, Wl1, bl1, Wr1, g1, be1,
           Wl2, bl2, Wr2, g2, be2, W1, b1, g3, be3, W2, b2):
    pad = EPAD - E
    src2 = jnp.concatenate([edge_index[0], jnp.zeros((pad,), jnp.int32)]
                           ).reshape(EPAD // CHUNK, CHUNK)
    dst2 = jnp.concatenate([edge_index[1], jnp.full((pad,), N, jnp.int32)]
                           ).reshape(EPAD // CHUNK, CHUNK)
    zrow = jnp.zeros((OCH, HA), jnp.float32)

    r2 = lambda a: a.reshape(1, -1)

    ha = _proj(x, W0, r2(b0), r2(g0), r2(be0))
    p1 = _sc_segsum(ha, src2, dst2, zrow)
    ha = _combine(p1, ha, Wl1, r2(bl1), Wr1, r2(g1), r2(be1))
    p2 = _sc_segsum(ha, src2, dst2, zrow)
    return _final(p2, ha, Wl2, r2(bl2), Wr2, r2(g2), r2(be2),
                  W1, r2(b1), g3.reshape(1, -1), r2(be3), W2, r2(b2))
